# baseline (speedup 1.0000x reference)
import functools

import jax
import jax.numpy as jnp
from jax import lax
from jax.experimental import pallas as pl
from jax.experimental.pallas import tpu as pltpu
from jax.experimental.pallas import tpu_sc as plsc

_N = 10000
_E = 320000
_G = 64

_NC = 2
_NS = 16
_NW = _NC * _NS
_EW = _E // _NW
_CHUNK = 80
_NCHUNK = _EW // _CHUNK
_ECH = _E // _CHUNK
_CHUNKL = 125
_NCHUNKL = _EW // _CHUNKL
_ECHL = _E // _CHUNKL
_ROWS_W = _N // _NS
_DEGW = 16

_ROWBLK = 2000
_NBLK = _N // _ROWBLK

_sc_mesh = plsc.VectorSubcoreMesh(core_axis_name="c", subcore_axis_name="s")
_sc_params = pltpu.CompilerParams(use_tc_tiling_on_sc=False)


def _make_edge_aggregate(feat, chunk, nbuf, dist):
    nchunk = _EW // chunk
    ech = _E // chunk

    @functools.partial(
        pl.kernel,
        out_type=jax.ShapeDtypeStruct((_NC * _N, feat), jnp.float32),
        mesh=_sc_mesh,
        scratch_types=(
            [pltpu.VMEM((nchunk, chunk), jnp.int32)] * 2 +
            [pltpu.VMEM((chunk, feat), jnp.float32)] * nbuf +
            [pltpu.VMEM_SHARED((_N, feat), jnp.float32)] +
            [pltpu.SemaphoreType.DMA] * (2 * nbuf)
        ),
        compiler_params=_sc_params,
    )
    def agg(y_hbm, ei_hbm, out_hbm, src2, dst2, *rest):
        rows = rest[:nbuf]
        acc_sh = rest[nbuf]
        gsem = rest[nbuf + 1:2 * nbuf + 1]
        ssem = rest[2 * nbuf + 1:]

        core = lax.axis_index("c")
        sid = lax.axis_index("s")
        wid = core * _NS + sid
        row0 = sid * _ROWS_W

        crow = wid * nchunk
        pltpu.sync_copy(ei_hbm.at[pl.ds(crow, nchunk)], src2)
        pltpu.sync_copy(ei_hbm.at[pl.ds(ech + crow, nchunk)], dst2)
        pltpu.sync_copy(y_hbm.at[pl.ds(row0, _ROWS_W)],
                        acc_sh.at[pl.ds(row0, _ROWS_W)])

        plsc.subcore_barrier()

        def gather(c, k):
            pltpu.async_copy(y_hbm.at[src2.at[c]], rows[k], gsem[k])

        def slot(c, k, swait, prefetch):
            pltpu.make_async_copy(y_hbm.at[src2.at[c]], rows[k],
                                  gsem[k]).wait()
            pltpu.async_copy(rows[k], acc_sh.at[dst2.at[c]], ssem[k],
                             add=True)
            if prefetch:
                kp = (k + dist) % nbuf
                if swait:
                    pltpu.make_async_copy(
                        rows[kp], acc_sh.at[dst2.at[c + dist - nbuf]],
                        ssem[kp]).wait()
                gather(c + dist, kp)

        for c in range(dist):
            gather(c, c)
        for c in range(nbuf - dist):
            slot(c, c % nbuf, False, True)

        start = nbuf - dist
        iters = (nchunk - nbuf) // nbuf
        cov = iters * nbuf

        @pl.loop(start, start + cov, step=nbuf)
        def _(c):
            for i in range(nbuf):
                slot(c + i, (start + i) % nbuf, True, True)

        for c in range(start + cov, nchunk - dist):
            slot(c, c % nbuf, True, True)
        for c in range(nchunk - dist, nchunk):
            slot(c, c % nbuf, False, False)

        for c in range(nchunk - nbuf, nchunk):
            pltpu.make_async_copy(rows[c % nbuf], acc_sh.at[dst2.at[c]],
                                  ssem[c % nbuf]).wait()

        plsc.subcore_barrier()
        pltpu.sync_copy(acc_sh.at[pl.ds(row0, _ROWS_W)],
                        out_hbm.at[pl.ds(core * _N + row0, _ROWS_W)])

    return agg


_agg128 = _make_edge_aggregate(128, _CHUNK, 3, 2)
_agg64 = _make_edge_aggregate(64, _CHUNKL, 5, 3)


@functools.partial(
    pl.kernel,
    out_type=jax.ShapeDtypeStruct((_NC * _N, _DEGW), jnp.float32),
    mesh=_sc_mesh,
    scratch_types=[
        pltpu.VMEM((_NCHUNKL, _CHUNKL), jnp.int32),
        pltpu.VMEM((_CHUNKL, _DEGW), jnp.float32),
        pltpu.VMEM((_ROWS_W, _DEGW), jnp.float32),
        pltpu.VMEM_SHARED((_N, _DEGW), jnp.float32),
        pltpu.SemaphoreType.DMA,
    ],
    compiler_params=_sc_params,
)
def _deg_count(ei_hbm, out_hbm, dst2, ones_v, seed_v, acc_sh, sem):
    core = lax.axis_index("c")
    sid = lax.axis_index("s")
    wid = core * _NS + sid
    row0 = sid * _ROWS_W

    pltpu.sync_copy(ei_hbm.at[pl.ds(_ECHL + wid * _NCHUNKL, _NCHUNKL)],
                    dst2)

    half = jnp.full((16,), 0.5, jnp.float32)
    one = jnp.ones((16,), jnp.float32)

    @pl.loop(0, _ROWS_W)
    def _(r):
        seed_v[r, :] = half

    @pl.loop(0, _CHUNKL)
    def _(r):
        ones_v[r, :] = one

    pltpu.sync_copy(seed_v, acc_sh.at[pl.ds(row0, _ROWS_W)])
    plsc.subcore_barrier()

    @pl.loop(0, _NCHUNKL, step=10)
    def _(j):
        for b in range(10):
            pltpu.async_copy(ones_v, acc_sh.at[dst2.at[j + b]], sem,
                             add=True)
        for b in range(10):
            pltpu.make_async_copy(ones_v, acc_sh.at[dst2.at[j + b]],
                                  sem).wait()

    plsc.subcore_barrier()
    pltpu.sync_copy(acc_sh.at[pl.ds(row0, _ROWS_W)],
                    out_hbm.at[pl.ds(core * _N + row0, _ROWS_W)])


def _mm1_body(x_ref, w_ref, o_ref):
    o_ref[...] = jnp.dot(x_ref[...], w_ref[...],
                         preferred_element_type=jnp.float32)


_mm1 = pl.pallas_call(
    _mm1_body,
    grid=(_NBLK,),
    in_specs=[pl.BlockSpec((_ROWBLK, 128), lambda i: (i, 0)),
              pl.BlockSpec((128, 128), lambda i: (0, 0))],
    out_specs=pl.BlockSpec((_ROWBLK, 128), lambda i: (i, 0)),
    out_shape=jax.ShapeDtypeStruct((_N, 128), jnp.float32),
)


def _scale_body(dega_ref, degb_ref, xw_ref, y_ref, dinv_ref):
    d = dega_ref[:, 0:1] + degb_ref[:, 0:1]
    dinv = lax.rsqrt(d)
    dinv_ref[...] = dinv
    y_ref[...] = xw_ref[...] * dinv


_scale = pl.pallas_call(
    _scale_body,
    grid=(_NBLK,),
    in_specs=[pl.BlockSpec((_ROWBLK, _DEGW), lambda i: (i, 0)),
              pl.BlockSpec((_ROWBLK, _DEGW), lambda i: (_N // _ROWBLK + i, 0)),
              pl.BlockSpec((_ROWBLK, 128), lambda i: (i, 0))],
    out_specs=[pl.BlockSpec((_ROWBLK, 128), lambda i: (i, 0)),
               pl.BlockSpec((_ROWBLK, 1), lambda i: (i, 0))],
    out_shape=[jax.ShapeDtypeStruct((_N, 128), jnp.float32),
               jax.ShapeDtypeStruct((_N, 1), jnp.float32)],
)


def _layer2_body(sa_ref, sb_ref, y1_ref, dinv_ref, b1_ref, w2_ref, y2_ref):
    dinv = dinv_ref[...]
    s = sa_ref[...] + sb_ref[...] - y1_ref[...]
    h1 = jnp.maximum(dinv * s + b1_ref[...], 0.0)
    xw2 = jnp.dot(h1, w2_ref[...], preferred_element_type=jnp.float32)
    y2_ref[...] = xw2 * dinv


_layer2 = pl.pallas_call(
    _layer2_body,
    grid=(_NBLK,),
    in_specs=[pl.BlockSpec((_ROWBLK, 128), lambda i: (i, 0)),
              pl.BlockSpec((_ROWBLK, 128), lambda i: (_N // _ROWBLK + i, 0)),
              pl.BlockSpec((_ROWBLK, 128), lambda i: (i, 0)),
              pl.BlockSpec((_ROWBLK, 1), lambda i: (i, 0)),
              pl.BlockSpec((1, 128), lambda i: (0, 0)),
              pl.BlockSpec((128, 64), lambda i: (0, 0))],
    out_specs=pl.BlockSpec((_ROWBLK, 64), lambda i: (i, 0)),
    out_shape=jax.ShapeDtypeStruct((_N, 64), jnp.float32),
)


def _final_body(sa_ref, sb_ref, y2_ref, dinv_ref, b2_ref, batch_ref,
                gamma_ref, beta_ref, wo1_ref, bo1_ref, wo2_ref, bo2_ref,
                out_ref, h_ref, pooled_acc):
    i = pl.program_id(0)
    dinv = dinv_ref[...]
    s = sa_ref[...] + sb_ref[...] - y2_ref[...]
    h2 = jnp.maximum(dinv * s + b2_ref[...], 0.0)
    seg = batch_ref[0]
    onehot_t = (seg == lax.broadcasted_iota(jnp.int32, (_G, _ROWBLK), 0))
    contrib = jnp.dot(onehot_t.astype(jnp.bfloat16),
                      h2.astype(jnp.bfloat16),
                      preferred_element_type=jnp.float32)

    @pl.when(i == 0)
    def _():
        pooled_acc[...] = jnp.zeros_like(pooled_acc)

    pooled_acc[...] += contrib

    @pl.when(i == pl.num_programs(0) - 1)
    def _():
        pooled = pooled_acc[...]
        mean = jnp.mean(pooled, axis=0, keepdims=True)
        var = jnp.mean((pooled - mean) ** 2, axis=0, keepdims=True)
        xn = (pooled - mean) / jnp.sqrt(var + 1e-5) * gamma_ref[...] \
            + beta_ref[...]
        t = jnp.maximum(
            jnp.dot(xn, wo1_ref[...], preferred_element_type=jnp.float32)
            + bo1_ref[...], 0.0)
        out_ref[...] = (
            jnp.dot(t, wo2_ref[...], preferred_element_type=jnp.float32)
            + bo2_ref[...])
        h_ref[...] = pooled


_final = pl.pallas_call(
    _final_body,
    grid=(_NBLK,),
    in_specs=[pl.BlockSpec((_ROWBLK, 64), lambda i: (i, 0)),
              pl.BlockSpec((_ROWBLK, 64), lambda i: (_N // _ROWBLK + i, 0)),
              pl.BlockSpec((_ROWBLK, 64), lambda i: (i, 0)),
              pl.BlockSpec((_ROWBLK, 1), lambda i: (i, 0)),
              pl.BlockSpec((1, 64), lambda i: (0, 0)),
              pl.BlockSpec((1, 1, _ROWBLK), lambda i: (i, 0, 0)),
              pl.BlockSpec((1, 64), lambda i: (0, 0)),
              pl.BlockSpec((1, 64), lambda i: (0, 0)),
              pl.BlockSpec((64, 24), lambda i: (0, 0)),
              pl.BlockSpec((1, 24), lambda i: (0, 0)),
              pl.BlockSpec((24, 1), lambda i: (0, 0)),
              pl.BlockSpec((1, 1), lambda i: (0, 0))],
    out_specs=[pl.BlockSpec((_G, 1), lambda i: (0, 0)),
               pl.BlockSpec((_G, _G), lambda i: (0, 0))],
    out_shape=[jax.ShapeDtypeStruct((_G, 1), jnp.float32),
               jax.ShapeDtypeStruct((_G, _G), jnp.float32)],
    scratch_shapes=[pltpu.VMEM((_G, _G), jnp.float32)],
)


def kernel(x, edge_index, batch, W1, b1, W2, b2, gamma, beta, Wo1, bo1, Wo2,
           bo2):
    ei80 = edge_index.reshape(2 * _ECH, _CHUNK)
    ei125 = edge_index.reshape(2 * _ECHL, _CHUNKL)

    degp = _deg_count(ei125)
    xw1 = _mm1(x, W1)
    y1, dinv = _scale(degp, degp, xw1)

    s1p = _agg128(y1, ei80)
    y2 = _layer2(s1p, s1p, y1, dinv, b1.reshape(1, 128), W2)

    s2p = _agg64(y2, ei125)
    out, h = _final(s2p, s2p, y2, dinv, b2.reshape(1, 64),
                    batch.reshape(_NBLK, 1, _ROWBLK), gamma.reshape(1, 64),
                    beta.reshape(1, 64), Wo1, bo1.reshape(1, 24), Wo2,
                    bo2.reshape(1, 1))
    return (out, h)


# speedup vs baseline: 1.0849x; 1.0849x over previous
import functools

import jax
import jax.numpy as jnp
from jax import lax
from jax.experimental import pallas as pl
from jax.experimental.pallas import tpu as pltpu
from jax.experimental.pallas import tpu_sc as plsc

_N = 10000
_E = 320000
_G = 64

_NC = 2
_NS = 16
_NW = _NC * _NS
_EW = _E // _NW
_CHUNK = 80
_NCHUNK = _EW // _CHUNK
_ECH = _E // _CHUNK
_CHUNKL = 125
_NCHUNKL = _EW // _CHUNKL
_ECHL = _E // _CHUNKL
_ROWS_W = _N // _NS
_DEGW = 16

_ROWBLK = 2000
_NBLK = _N // _ROWBLK

_sc_mesh = plsc.VectorSubcoreMesh(core_axis_name="c", subcore_axis_name="s")
_sc_params = pltpu.CompilerParams(use_tc_tiling_on_sc=False)


def _make_edge_aggregate(feat, chunk, nbuf, dist):
    nchunk = _EW // chunk
    ech = _E // chunk

    @functools.partial(
        pl.kernel,
        out_type=jax.ShapeDtypeStruct((_NC * _N, feat), jnp.bfloat16),
        mesh=_sc_mesh,
        scratch_types=(
            [pltpu.VMEM((nchunk, chunk), jnp.int32)] * 2 +
            [pltpu.VMEM((chunk, feat), jnp.bfloat16)] * nbuf +
            [pltpu.VMEM_SHARED((_N, feat), jnp.bfloat16)] +
            [pltpu.SemaphoreType.DMA] * (2 * nbuf)
        ),
        compiler_params=_sc_params,
    )
    def agg(y_hbm, ei_hbm, out_hbm, src2, dst2, *rest):
        rows = rest[:nbuf]
        acc_sh = rest[nbuf]
        gsem = rest[nbuf + 1:2 * nbuf + 1]
        ssem = rest[2 * nbuf + 1:]

        core = lax.axis_index("c")
        sid = lax.axis_index("s")
        wid = core * _NS + sid
        row0 = sid * _ROWS_W

        crow = wid * nchunk
        pltpu.sync_copy(ei_hbm.at[pl.ds(crow, nchunk)], src2)
        pltpu.sync_copy(ei_hbm.at[pl.ds(ech + crow, nchunk)], dst2)
        pltpu.sync_copy(y_hbm.at[pl.ds(row0, _ROWS_W)],
                        acc_sh.at[pl.ds(row0, _ROWS_W)])

        plsc.subcore_barrier()

        def gather(c, k):
            pltpu.async_copy(y_hbm.at[src2.at[c]], rows[k], gsem[k])

        def slot(c, k, swait, prefetch):
            pltpu.make_async_copy(y_hbm.at[src2.at[c]], rows[k],
                                  gsem[k]).wait()
            pltpu.async_copy(rows[k], acc_sh.at[dst2.at[c]], ssem[k],
                             add=True)
            if prefetch:
                kp = (k + dist) % nbuf
                if swait:
                    pltpu.make_async_copy(
                        rows[kp], acc_sh.at[dst2.at[c + dist - nbuf]],
                        ssem[kp]).wait()
                gather(c + dist, kp)

        for c in range(dist):
            gather(c, c)
        for c in range(nbuf - dist):
            slot(c, c % nbuf, False, True)

        start = nbuf - dist
        iters = (nchunk - nbuf) // nbuf
        cov = iters * nbuf

        @pl.loop(start, start + cov, step=nbuf)
        def _(c):
            for i in range(nbuf):
                slot(c + i, (start + i) % nbuf, True, True)

        for c in range(start + cov, nchunk - dist):
            slot(c, c % nbuf, True, True)
        for c in range(nchunk - dist, nchunk):
            slot(c, c % nbuf, False, False)

        for c in range(nchunk - nbuf, nchunk):
            pltpu.make_async_copy(rows[c % nbuf], acc_sh.at[dst2.at[c]],
                                  ssem[c % nbuf]).wait()

        plsc.subcore_barrier()
        pltpu.sync_copy(acc_sh.at[pl.ds(row0, _ROWS_W)],
                        out_hbm.at[pl.ds(core * _N + row0, _ROWS_W)])

    return agg


_agg128 = _make_edge_aggregate(128, _CHUNK, 3, 2)
_agg64 = _make_edge_aggregate(64, _CHUNKL, 5, 3)


@functools.partial(
    pl.kernel,
    out_type=jax.ShapeDtypeStruct((_NC * _N, _DEGW), jnp.float32),
    mesh=_sc_mesh,
    scratch_types=[
        pltpu.VMEM((_NCHUNKL, _CHUNKL), jnp.int32),
        pltpu.VMEM((_CHUNKL, _DEGW), jnp.float32),
        pltpu.VMEM((_ROWS_W, _DEGW), jnp.float32),
        pltpu.VMEM_SHARED((_N, _DEGW), jnp.float32),
        pltpu.SemaphoreType.DMA,
    ],
    compiler_params=_sc_params,
)
def _deg_count(ei_hbm, out_hbm, dst2, ones_v, seed_v, acc_sh, sem):
    core = lax.axis_index("c")
    sid = lax.axis_index("s")
    wid = core * _NS + sid
    row0 = sid * _ROWS_W

    pltpu.sync_copy(ei_hbm.at[pl.ds(_ECHL + wid * _NCHUNKL, _NCHUNKL)],
                    dst2)

    half = jnp.full((16,), 0.5, jnp.float32)
    one = jnp.ones((16,), jnp.float32)

    @pl.loop(0, _ROWS_W)
    def _(r):
        seed_v[r, :] = half

    @pl.loop(0, _CHUNKL)
    def _(r):
        ones_v[r, :] = one

    pltpu.sync_copy(seed_v, acc_sh.at[pl.ds(row0, _ROWS_W)])
    plsc.subcore_barrier()

    @pl.loop(0, _NCHUNKL, step=10)
    def _(j):
        for b in range(10):
            pltpu.async_copy(ones_v, acc_sh.at[dst2.at[j + b]], sem,
                             add=True)
        for b in range(10):
            pltpu.make_async_copy(ones_v, acc_sh.at[dst2.at[j + b]],
                                  sem).wait()

    plsc.subcore_barrier()
    pltpu.sync_copy(acc_sh.at[pl.ds(row0, _ROWS_W)],
                    out_hbm.at[pl.ds(core * _N + row0, _ROWS_W)])


def _mm1_body(x_ref, w_ref, o_ref):
    o_ref[...] = jnp.dot(x_ref[...], w_ref[...],
                         preferred_element_type=jnp.float32)


_mm1 = pl.pallas_call(
    _mm1_body,
    grid=(_NBLK,),
    in_specs=[pl.BlockSpec((_ROWBLK, 128), lambda i: (i, 0)),
              pl.BlockSpec((128, 128), lambda i: (0, 0))],
    out_specs=pl.BlockSpec((_ROWBLK, 128), lambda i: (i, 0)),
    out_shape=jax.ShapeDtypeStruct((_N, 128), jnp.float32),
)


def _scale_body(dega_ref, degb_ref, xw_ref, y_ref, dinv_ref):
    d = dega_ref[:, 0:1] + degb_ref[:, 0:1]
    dinv = lax.rsqrt(d)
    dinv_ref[...] = dinv
    y_ref[...] = (xw_ref[...] * dinv).astype(jnp.bfloat16)


_scale = pl.pallas_call(
    _scale_body,
    grid=(_NBLK,),
    in_specs=[pl.BlockSpec((_ROWBLK, _DEGW), lambda i: (i, 0)),
              pl.BlockSpec((_ROWBLK, _DEGW), lambda i: (_N // _ROWBLK + i, 0)),
              pl.BlockSpec((_ROWBLK, 128), lambda i: (i, 0))],
    out_specs=[pl.BlockSpec((_ROWBLK, 128), lambda i: (i, 0)),
               pl.BlockSpec((_ROWBLK, 1), lambda i: (i, 0))],
    out_shape=[jax.ShapeDtypeStruct((_N, 128), jnp.bfloat16),
               jax.ShapeDtypeStruct((_N, 1), jnp.float32)],
)


def _layer2_body(sa_ref, sb_ref, y1_ref, dinv_ref, b1_ref, w2_ref, y2_ref):
    dinv = dinv_ref[...]
    s = (sa_ref[...].astype(jnp.float32) + sb_ref[...].astype(jnp.float32)
         - y1_ref[...].astype(jnp.float32))
    h1 = jnp.maximum(dinv * s + b1_ref[...], 0.0)
    xw2 = jnp.dot(h1, w2_ref[...], preferred_element_type=jnp.float32)
    y2_ref[...] = (xw2 * dinv).astype(jnp.bfloat16)


_layer2 = pl.pallas_call(
    _layer2_body,
    grid=(_NBLK,),
    in_specs=[pl.BlockSpec((_ROWBLK, 128), lambda i: (i, 0)),
              pl.BlockSpec((_ROWBLK, 128), lambda i: (_N // _ROWBLK + i, 0)),
              pl.BlockSpec((_ROWBLK, 128), lambda i: (i, 0)),
              pl.BlockSpec((_ROWBLK, 1), lambda i: (i, 0)),
              pl.BlockSpec((1, 128), lambda i: (0, 0)),
              pl.BlockSpec((128, 64), lambda i: (0, 0))],
    out_specs=pl.BlockSpec((_ROWBLK, 64), lambda i: (i, 0)),
    out_shape=jax.ShapeDtypeStruct((_N, 64), jnp.bfloat16),
)


def _final_body(sa_ref, sb_ref, y2_ref, dinv_ref, b2_ref, batch_ref,
                gamma_ref, beta_ref, wo1_ref, bo1_ref, wo2_ref, bo2_ref,
                out_ref, h_ref, pooled_acc):
    i = pl.program_id(0)
    dinv = dinv_ref[...]
    s = (sa_ref[...].astype(jnp.float32) + sb_ref[...].astype(jnp.float32)
         - y2_ref[...].astype(jnp.float32))
    h2 = jnp.maximum(dinv * s + b2_ref[...], 0.0)
    seg = batch_ref[0]
    onehot_t = (seg == lax.broadcasted_iota(jnp.int32, (_G, _ROWBLK), 0))
    contrib = jnp.dot(onehot_t.astype(jnp.bfloat16),
                      h2.astype(jnp.bfloat16),
                      preferred_element_type=jnp.float32)

    @pl.when(i == 0)
    def _():
        pooled_acc[...] = jnp.zeros_like(pooled_acc)

    pooled_acc[...] += contrib

    @pl.when(i == pl.num_programs(0) - 1)
    def _():
        pooled = pooled_acc[...]
        mean = jnp.mean(pooled, axis=0, keepdims=True)
        var = jnp.mean((pooled - mean) ** 2, axis=0, keepdims=True)
        xn = (pooled - mean) / jnp.sqrt(var + 1e-5) * gamma_ref[...] \
            + beta_ref[...]
        t = jnp.maximum(
            jnp.dot(xn, wo1_ref[...], preferred_element_type=jnp.float32)
            + bo1_ref[...], 0.0)
        out_ref[...] = (
            jnp.dot(t, wo2_ref[...], preferred_element_type=jnp.float32)
            + bo2_ref[...])
        h_ref[...] = pooled


_final = pl.pallas_call(
    _final_body,
    grid=(_NBLK,),
    in_specs=[pl.BlockSpec((_ROWBLK, 64), lambda i: (i, 0)),
              pl.BlockSpec((_ROWBLK, 64), lambda i: (_N // _ROWBLK + i, 0)),
              pl.BlockSpec((_ROWBLK, 64), lambda i: (i, 0)),
              pl.BlockSpec((_ROWBLK, 1), lambda i: (i, 0)),
              pl.BlockSpec((1, 64), lambda i: (0, 0)),
              pl.BlockSpec((1, 1, _ROWBLK), lambda i: (i, 0, 0)),
              pl.BlockSpec((1, 64), lambda i: (0, 0)),
              pl.BlockSpec((1, 64), lambda i: (0, 0)),
              pl.BlockSpec((64, 24), lambda i: (0, 0)),
              pl.BlockSpec((1, 24), lambda i: (0, 0)),
              pl.BlockSpec((24, 1), lambda i: (0, 0)),
              pl.BlockSpec((1, 1), lambda i: (0, 0))],
    out_specs=[pl.BlockSpec((_G, 1), lambda i: (0, 0)),
               pl.BlockSpec((_G, _G), lambda i: (0, 0))],
    out_shape=[jax.ShapeDtypeStruct((_G, 1), jnp.float32),
               jax.ShapeDtypeStruct((_G, _G), jnp.float32)],
    scratch_shapes=[pltpu.VMEM((_G, _G), jnp.float32)],
)


def kernel(x, edge_index, batch, W1, b1, W2, b2, gamma, beta, Wo1, bo1, Wo2,
           bo2):
    ei80 = edge_index.reshape(2 * _ECH, _CHUNK)
    ei125 = edge_index.reshape(2 * _ECHL, _CHUNKL)

    degp = _deg_count(ei125)
    xw1 = _mm1(x, W1)
    y1, dinv = _scale(degp, degp, xw1)

    s1p = _agg128(y1, ei80)
    y2 = _layer2(s1p, s1p, y1, dinv, b1.reshape(1, 128), W2)

    s2p = _agg64(y2, ei125)
    out, h = _final(s2p, s2p, y2, dinv, b2.reshape(1, 64),
                    batch.reshape(_NBLK, 1, _ROWBLK), gamma.reshape(1, 64),
                    beta.reshape(1, 64), Wo1, bo1.reshape(1, 24), Wo2,
                    bo2.reshape(1, 1))
    return (out, h)


# speedup vs baseline: 1.0952x; 1.0095x over previous
import functools

import jax
import jax.numpy as jnp
from jax import lax
from jax.experimental import pallas as pl
from jax.experimental.pallas import tpu as pltpu
from jax.experimental.pallas import tpu_sc as plsc

_N = 10000
_E = 320000
_G = 64

_NC = 2
_NS = 16
_NW = _NC * _NS
_EW = _E // _NW
_CHUNKL = 125
_NCHUNKL = _EW // _CHUNKL
_ECHL = _E // _CHUNKL
_ROWS_W = _N // _NS
_DEGW = 16

_ROWBLK = 2000
_NBLK = _N // _ROWBLK

_sc_mesh = plsc.VectorSubcoreMesh(core_axis_name="c", subcore_axis_name="s")
_sc_params = pltpu.CompilerParams(use_tc_tiling_on_sc=False)


def _make_edge_aggregate(feat, chunk, nbuf, dist):
    nchunk = _EW // chunk

    @functools.partial(
        pl.kernel,
        out_type=jax.ShapeDtypeStruct((_NC * _N, feat), jnp.bfloat16),
        mesh=_sc_mesh,
        scratch_types=(
            [pltpu.VMEM((nchunk, chunk), jnp.int32)] * 2 +
            [pltpu.VMEM((chunk, feat), jnp.bfloat16)] * nbuf +
            [pltpu.VMEM_SHARED((_N, feat), jnp.bfloat16)] +
            [pltpu.SemaphoreType.DMA] * (2 * nbuf)
        ),
        compiler_params=_sc_params,
    )
    def agg(y_hbm, src_hbm, dst_hbm, out_hbm, src2, dst2, *rest):
        rows = rest[:nbuf]
        acc_sh = rest[nbuf]
        gsem = rest[nbuf + 1:2 * nbuf + 1]
        ssem = rest[2 * nbuf + 1:]

        core = lax.axis_index("c")
        sid = lax.axis_index("s")
        wid = core * _NS + sid
        row0 = sid * _ROWS_W

        crow = wid * nchunk
        pltpu.sync_copy(src_hbm.at[pl.ds(crow, nchunk)], src2)
        pltpu.sync_copy(dst_hbm.at[pl.ds(crow, nchunk)], dst2)
        pltpu.sync_copy(y_hbm.at[pl.ds(row0, _ROWS_W)],
                        acc_sh.at[pl.ds(row0, _ROWS_W)])

        plsc.subcore_barrier()

        def gather(c, k):
            pltpu.async_copy(y_hbm.at[src2.at[c]], rows[k], gsem[k])

        def slot(c, k, swait, prefetch):
            pltpu.make_async_copy(y_hbm.at[src2.at[c]], rows[k],
                                  gsem[k]).wait()
            pltpu.async_copy(rows[k], acc_sh.at[dst2.at[c]], ssem[k],
                             add=True)
            if prefetch:
                kp = (k + dist) % nbuf
                if swait:
                    pltpu.make_async_copy(
                        rows[kp], acc_sh.at[dst2.at[c + dist - nbuf]],
                        ssem[kp]).wait()
                gather(c + dist, kp)

        for c in range(dist):
            gather(c, c)
        for c in range(nbuf - dist):
            slot(c, c % nbuf, False, True)

        start = nbuf - dist
        iters = (nchunk - nbuf) // nbuf
        cov = iters * nbuf

        @pl.loop(start, start + cov, step=nbuf)
        def _(c):
            for i in range(nbuf):
                slot(c + i, (start + i) % nbuf, True, True)

        for c in range(start + cov, nchunk - dist):
            slot(c, c % nbuf, True, True)
        for c in range(nchunk - dist, nchunk):
            slot(c, c % nbuf, False, False)

        for c in range(nchunk - nbuf, nchunk):
            pltpu.make_async_copy(rows[c % nbuf], acc_sh.at[dst2.at[c]],
                                  ssem[c % nbuf]).wait()

        plsc.subcore_barrier()
        pltpu.sync_copy(acc_sh.at[pl.ds(row0, _ROWS_W)],
                        out_hbm.at[pl.ds(core * _N + row0, _ROWS_W)])

    return agg


_agg128 = _make_edge_aggregate(128, _CHUNKL, 3, 2)
_agg64 = _make_edge_aggregate(64, _CHUNKL, 5, 3)


@functools.partial(
    pl.kernel,
    out_type=jax.ShapeDtypeStruct((_NC * _N, _DEGW), jnp.float32),
    mesh=_sc_mesh,
    scratch_types=[
        pltpu.VMEM((_NCHUNKL, _CHUNKL), jnp.int32),
        pltpu.VMEM((_CHUNKL, _DEGW), jnp.float32),
        pltpu.VMEM((_ROWS_W, _DEGW), jnp.float32),
        pltpu.VMEM_SHARED((_N, _DEGW), jnp.float32),
        pltpu.SemaphoreType.DMA,
    ],
    compiler_params=_sc_params,
)
def _deg_count(dst_hbm, out_hbm, dst2, ones_v, seed_v, acc_sh, sem):
    core = lax.axis_index("c")
    sid = lax.axis_index("s")
    wid = core * _NS + sid
    row0 = sid * _ROWS_W

    pltpu.sync_copy(dst_hbm.at[pl.ds(wid * _NCHUNKL, _NCHUNKL)], dst2)

    half = jnp.full((16,), 0.5, jnp.float32)
    one = jnp.ones((16,), jnp.float32)

    @pl.loop(0, _ROWS_W)
    def _(r):
        seed_v[r, :] = half

    @pl.loop(0, _CHUNKL)
    def _(r):
        ones_v[r, :] = one

    pltpu.sync_copy(seed_v, acc_sh.at[pl.ds(row0, _ROWS_W)])
    plsc.subcore_barrier()

    @pl.loop(0, _NCHUNKL, step=10)
    def _(j):
        for b in range(10):
            pltpu.async_copy(ones_v, acc_sh.at[dst2.at[j + b]], sem,
                             add=True)
        for b in range(10):
            pltpu.make_async_copy(ones_v, acc_sh.at[dst2.at[j + b]],
                                  sem).wait()

    plsc.subcore_barrier()
    pltpu.sync_copy(acc_sh.at[pl.ds(row0, _ROWS_W)],
                    out_hbm.at[pl.ds(core * _N + row0, _ROWS_W)])


def _mm1_body(x_ref, w_ref, o_ref):
    o_ref[...] = jnp.dot(x_ref[...], w_ref[...],
                         preferred_element_type=jnp.float32)


_mm1 = pl.pallas_call(
    _mm1_body,
    grid=(_NBLK,),
    in_specs=[pl.BlockSpec((_ROWBLK, 128), lambda i: (i, 0)),
              pl.BlockSpec((128, 128), lambda i: (0, 0))],
    out_specs=pl.BlockSpec((_ROWBLK, 128), lambda i: (i, 0)),
    out_shape=jax.ShapeDtypeStruct((_N, 128), jnp.float32),
)


def _scale_body(dega_ref, degb_ref, xw_ref, y_ref, dinv_ref):
    d = dega_ref[:, 0:1] + degb_ref[:, 0:1]
    dinv = lax.rsqrt(d)
    dinv_ref[...] = dinv
    y_ref[...] = (xw_ref[...] * dinv).astype(jnp.bfloat16)


_scale = pl.pallas_call(
    _scale_body,
    grid=(_NBLK,),
    in_specs=[pl.BlockSpec((_ROWBLK, _DEGW), lambda i: (i, 0)),
              pl.BlockSpec((_ROWBLK, _DEGW), lambda i: (_N // _ROWBLK + i, 0)),
              pl.BlockSpec((_ROWBLK, 128), lambda i: (i, 0))],
    out_specs=[pl.BlockSpec((_ROWBLK, 128), lambda i: (i, 0)),
               pl.BlockSpec((_ROWBLK, 1), lambda i: (i, 0))],
    out_shape=[jax.ShapeDtypeStruct((_N, 128), jnp.bfloat16),
               jax.ShapeDtypeStruct((_N, 1), jnp.float32)],
)


def _layer2_body(sa_ref, sb_ref, y1_ref, dinv_ref, b1_ref, w2_ref, y2_ref):
    dinv = dinv_ref[...]
    s = (sa_ref[...].astype(jnp.float32) + sb_ref[...].astype(jnp.float32)
         - y1_ref[...].astype(jnp.float32))
    h1 = jnp.maximum(dinv * s + b1_ref[...], 0.0)
    xw2 = jnp.dot(h1, w2_ref[...], preferred_element_type=jnp.float32)
    y2_ref[...] = (xw2 * dinv).astype(jnp.bfloat16)


_layer2 = pl.pallas_call(
    _layer2_body,
    grid=(_NBLK,),
    in_specs=[pl.BlockSpec((_ROWBLK, 128), lambda i: (i, 0)),
              pl.BlockSpec((_ROWBLK, 128), lambda i: (_N // _ROWBLK + i, 0)),
              pl.BlockSpec((_ROWBLK, 128), lambda i: (i, 0)),
              pl.BlockSpec((_ROWBLK, 1), lambda i: (i, 0)),
              pl.BlockSpec((1, 128), lambda i: (0, 0)),
              pl.BlockSpec((128, 64), lambda i: (0, 0))],
    out_specs=pl.BlockSpec((_ROWBLK, 64), lambda i: (i, 0)),
    out_shape=jax.ShapeDtypeStruct((_N, 64), jnp.bfloat16),
)


def _final_body(sa_ref, sb_ref, y2_ref, dinv_ref, b2_ref, batch_ref,
                gamma_ref, beta_ref, wo1_ref, bo1_ref, wo2_ref, bo2_ref,
                out_ref, h_ref, pooled_acc):
    i = pl.program_id(0)
    dinv = dinv_ref[...]
    s = (sa_ref[...].astype(jnp.float32) + sb_ref[...].astype(jnp.float32)
         - y2_ref[...].astype(jnp.float32))
    h2 = jnp.maximum(dinv * s + b2_ref[...], 0.0)
    seg = batch_ref[0]
    onehot_t = (seg == lax.broadcasted_iota(jnp.int32, (_G, _ROWBLK), 0))
    contrib = jnp.dot(onehot_t.astype(jnp.bfloat16),
                      h2.astype(jnp.bfloat16),
                      preferred_element_type=jnp.float32)

    @pl.when(i == 0)
    def _():
        pooled_acc[...] = jnp.zeros_like(pooled_acc)

    pooled_acc[...] += contrib

    @pl.when(i == pl.num_programs(0) - 1)
    def _():
        pooled = pooled_acc[...]
        mean = jnp.mean(pooled, axis=0, keepdims=True)
        var = jnp.mean((pooled - mean) ** 2, axis=0, keepdims=True)
        xn = (pooled - mean) / jnp.sqrt(var + 1e-5) * gamma_ref[...] \
            + beta_ref[...]
        t = jnp.maximum(
            jnp.dot(xn, wo1_ref[...], preferred_element_type=jnp.float32)
            + bo1_ref[...], 0.0)
        out_ref[...] = (
            jnp.dot(t, wo2_ref[...], preferred_element_type=jnp.float32)
            + bo2_ref[...])
        h_ref[...] = pooled


_final = pl.pallas_call(
    _final_body,
    grid=(_NBLK,),
    in_specs=[pl.BlockSpec((_ROWBLK, 64), lambda i: (i, 0)),
              pl.BlockSpec((_ROWBLK, 64), lambda i: (_N // _ROWBLK + i, 0)),
              pl.BlockSpec((_ROWBLK, 64), lambda i: (i, 0)),
              pl.BlockSpec((_ROWBLK, 1), lambda i: (i, 0)),
              pl.BlockSpec((1, 64), lambda i: (0, 0)),
              pl.BlockSpec((1, 1, _ROWBLK), lambda i: (i, 0, 0)),
              pl.BlockSpec((1, 64), lambda i: (0, 0)),
              pl.BlockSpec((1, 64), lambda i: (0, 0)),
              pl.BlockSpec((64, 24), lambda i: (0, 0)),
              pl.BlockSpec((1, 24), lambda i: (0, 0)),
              pl.BlockSpec((24, 1), lambda i: (0, 0)),
              pl.BlockSpec((1, 1), lambda i: (0, 0))],
    out_specs=[pl.BlockSpec((_G, 1), lambda i: (0, 0)),
               pl.BlockSpec((_G, _G), lambda i: (0, 0))],
    out_shape=[jax.ShapeDtypeStruct((_G, 1), jnp.float32),
               jax.ShapeDtypeStruct((_G, _G), jnp.float32)],
    scratch_shapes=[pltpu.VMEM((_G, _G), jnp.float32)],
)


def kernel(x, edge_index, batch, W1, b1, W2, b2, gamma, beta, Wo1, bo1, Wo2,
           bo2):
    src125 = edge_index[0].reshape(_ECHL, _CHUNKL)
    dst125 = edge_index[1].reshape(_ECHL, _CHUNKL)

    degp = _deg_count(dst125)
    xw1 = _mm1(x, W1)
    y1, dinv = _scale(degp, degp, xw1)

    s1p = _agg128(y1, src125, dst125)
    y2 = _layer2(s1p, s1p, y1, dinv, b1.reshape(1, 128), W2)

    s2p = _agg64(y2, src125, dst125)
    out, h = _final(s2p, s2p, y2, dinv, b2.reshape(1, 64),
                    batch.reshape(_NBLK, 1, _ROWBLK), gamma.reshape(1, 64),
                    beta.reshape(1, 64), Wo1, bo1.reshape(1, 24), Wo2,
                    bo2.reshape(1, 1))
    return (out, h)
